# baseline (speedup 1.0000x reference)
import functools

import jax
import jax.numpy as jnp
from jax import lax
from jax.experimental import pallas as pl
from jax.experimental.pallas import tpu as pltpu
from jax.experimental.pallas import tpu_sc as plsc


TILE = 2048
CH = 1024
E = 8
L = 16


def _logits_body(x_ref, gw_ref, l3_ref):
    xt = x_ref[...]
    gw = gw_ref[...]
    logits = lax.dot_general(gw, xt, (((1,), (1,)), ((), ())),
                             preferred_element_type=jnp.float32)
    for c in range(TILE // CH):
        l3_ref[c] = logits[:, c * CH:(c + 1) * CH]


def _sc_route_body(l_hbm, d_hbm, l_v, d_v):
    wid = lax.axis_index("s") * 2 + lax.axis_index("c")
    pltpu.sync_copy(l_hbm.at[wid], l_v)
    for j in range(CH // L):
        sl = pl.ds(j * L, L)
        lv = [l_v[e, sl] for e in range(E)]
        m1 = lv[0]
        for e in range(1, E):
            m1 = jnp.maximum(m1, lv[e])
        mask1 = [lv[e] == m1 for e in range(E)]
        neg = [jnp.where(mask1[e], -jnp.inf, lv[e]) for e in range(E)]
        m2 = neg[0]
        for e in range(1, E):
            m2 = jnp.maximum(m2, neg[e])
        e21 = jnp.exp(m2 - m1)
        w1 = 1.0 / (1.0 + e21)
        w2 = e21 * w1
        for e in range(E):
            d_v[e, sl] = jnp.where(mask1[e], w1,
                                   jnp.where(neg[e] == m2, w2, 0.0))
    pltpu.sync_copy(d_v, d_hbm.at[wid])


def _accum_body(x_ref, d3_ref, ei_ref):
    i = pl.program_id(0)
    xt = x_ref[...]
    dvec = jnp.concatenate([d3_ref[c] for c in range(TILE // CH)], axis=1)
    contrib = lax.dot_general(dvec, xt, (((1,), (0,)), ((), ())),
                              preferred_element_type=jnp.float32)

    @pl.when(i == 0)
    def _():
        ei_ref[...] = jnp.zeros_like(ei_ref)

    ei_ref[...] += contrib


def _combine_body(d3_ref, ei_ref, w_ref, b_ref, out_ref, y_s):
    i = pl.program_id(0)

    @pl.when(i == 0)
    def _():
        for e in range(E):
            row = lax.dot_general(ei_ref[e:e + 1, :], w_ref[e],
                                  (((1,), (1,)), ((), ())),
                                  preferred_element_type=jnp.float32)
            y_s[e:e + 1, :] = row + b_ref[e:e + 1, :]

    for c in range(TILE // CH):
        out_ref[pl.ds(c * CH, CH), :] = lax.dot_general(
            d3_ref[c], y_s[...], (((0,), (0,)), ((), ())),
            preferred_element_type=jnp.float32)


@jax.jit
def _moe(x, gate_W, expert_weights, expert_W, expert_b):
    B, S, D = x.shape
    T = B * S
    _, F, _ = expert_W.shape
    x_flat = x.reshape(T, D)
    gw = gate_W * expert_weights[:, None]
    n_tiles = T // TILE
    n_ch = T // CH

    l3 = pl.pallas_call(
        _logits_body,
        grid=(n_tiles,),
        in_specs=[
            pl.BlockSpec((TILE, D), lambda i: (i, 0)),
            pl.BlockSpec((E, D), lambda i: (0, 0)),
        ],
        out_specs=pl.BlockSpec((TILE // CH, E, CH), lambda i: (i, 0, 0)),
        out_shape=jax.ShapeDtypeStruct((n_ch, E, CH), jnp.float32),
    )(x_flat, gw)

    mesh = plsc.VectorSubcoreMesh(core_axis_name="c", subcore_axis_name="s")
    route = functools.partial(
        pl.kernel,
        mesh=mesh,
        out_type=jax.ShapeDtypeStruct((n_ch, E, CH), jnp.float32),
        scratch_types=[
            pltpu.VMEM((E, CH), jnp.float32),
            pltpu.VMEM((E, CH), jnp.float32),
        ],
    )(_sc_route_body)
    d3 = route(l3)

    ei = pl.pallas_call(
        _accum_body,
        grid=(n_tiles,),
        in_specs=[
            pl.BlockSpec((TILE, D), lambda i: (i, 0)),
            pl.BlockSpec((TILE // CH, E, CH), lambda i: (i, 0, 0)),
        ],
        out_specs=pl.BlockSpec((E, D), lambda i: (0, 0)),
        out_shape=jax.ShapeDtypeStruct((E, D), jnp.float32),
    )(x_flat, d3)

    out = pl.pallas_call(
        _combine_body,
        grid=(n_tiles,),
        in_specs=[
            pl.BlockSpec((TILE // CH, E, CH), lambda i: (i, 0, 0)),
            pl.BlockSpec((E, D), lambda i: (0, 0)),
            pl.BlockSpec((E, F, D), lambda i: (0, 0, 0)),
            pl.BlockSpec((E, F), lambda i: (0, 0)),
        ],
        out_specs=pl.BlockSpec((TILE, F), lambda i: (i, 0)),
        out_shape=jax.ShapeDtypeStruct((T, F), jnp.float32),
        scratch_shapes=[pltpu.VMEM((E, F), jnp.float32)],
    )(d3, ei, expert_W, expert_b)

    return out.reshape(B, S, F)


def kernel(x, gate_W, expert_weights, expert_W, expert_b):
    return _moe(x, gate_W, expert_weights, expert_W, expert_b)


# speedup vs baseline: 1.6427x; 1.6427x over previous
import functools

import jax
import jax.numpy as jnp
from jax import lax
from jax.experimental import pallas as pl
from jax.experimental.pallas import tpu as pltpu


TILE0 = 2048
TILE1 = 2048


def _body(n0, x_ref, gw_ref, w_hbm, b_ref, out_ref, dvec_s, ei_s, y_s, w_s, w_sem):
    s = pl.program_id(0)
    E = gw_ref.shape[0]

    @pl.when(s == 0)
    def _start_w():
        pltpu.make_async_copy(w_hbm, w_s, w_sem).start()

    @pl.when(s < n0)
    def _phase0():
        xt = x_ref[...]
        gw = gw_ref[...]
        logits = lax.dot_general(gw, xt, (((1,), (1,)), ((), ())),
                                 preferred_element_type=jnp.float32)
        m1 = jnp.max(logits, axis=0, keepdims=True)
        mask1 = logits == m1
        neg = jnp.where(mask1, -jnp.inf, logits)
        m2 = jnp.max(neg, axis=0, keepdims=True)
        mask2 = neg == m2
        e21 = jnp.exp(m2 - m1)
        w1 = 1.0 / (1.0 + e21)
        w2 = e21 * w1
        dvec = jnp.where(mask1, w1, jnp.where(mask2, w2, 0.0))
        dvec_s[:, pl.ds(s * TILE0, TILE0)] = dvec
        contrib = lax.dot_general(dvec, xt, (((1,), (0,)), ((), ())),
                                  preferred_element_type=jnp.float32)

        @pl.when(s == 0)
        def _():
            ei_s[...] = jnp.zeros_like(ei_s)

        ei_s[...] += contrib

    @pl.when(s == n0)
    def _expert():
        pltpu.make_async_copy(w_hbm, w_s, w_sem).wait()
        for e in range(E):
            row = lax.dot_general(ei_s[e:e + 1, :], w_s[e],
                                  (((1,), (1,)), ((), ())),
                                  preferred_element_type=jnp.float32)
            y_s[e:e + 1, :] = row + b_ref[e:e + 1, :]

    @pl.when(s >= n0)
    def _phase1():
        dvec = dvec_s[:, pl.ds((s - n0) * TILE1, TILE1)]
        out_ref[...] = lax.dot_general(dvec, y_s[...], (((0,), (0,)), ((), ())),
                                       preferred_element_type=jnp.float32)


@jax.jit
def _moe(x, gate_W, expert_weights, expert_W, expert_b):
    B, S, D = x.shape
    T = B * S
    E, F, _ = expert_W.shape
    x_flat = x.reshape(T, D)
    gw = gate_W * expert_weights[:, None]
    n0 = T // TILE0
    n1 = T // TILE1

    out = pl.pallas_call(
        functools.partial(_body, n0),
        grid=(n0 + n1,),
        in_specs=[
            pl.BlockSpec((TILE0, D), lambda s: (jnp.minimum(s, n0 - 1), 0)),
            pl.BlockSpec((E, D), lambda s: (0, 0)),
            pl.BlockSpec(memory_space=pl.ANY),
            pl.BlockSpec((E, F), lambda s: (0, 0)),
        ],
        out_specs=pl.BlockSpec((TILE1, F), lambda s: (jnp.maximum(s - n0, 0), 0)),
        out_shape=jax.ShapeDtypeStruct((T, F), jnp.float32),
        scratch_shapes=[
            pltpu.VMEM((E, T), jnp.float32),
            pltpu.VMEM((E, D), jnp.float32),
            pltpu.VMEM((E, F), jnp.float32),
            pltpu.VMEM((E, F, D), jnp.float32),
            pltpu.SemaphoreType.DMA,
        ],
    )(x_flat, gw, expert_W, expert_b)

    return out.reshape(B, S, F)


def kernel(x, gate_W, expert_weights, expert_W, expert_b):
    return _moe(x, gate_W, expert_weights, expert_W, expert_b)
